# baseline (speedup 1.0000x reference)
import functools
import math

import jax
import jax.numpy as jnp
from jax import lax
from jax.experimental import pallas as pl
from jax.experimental.pallas import tpu as pltpu
from jax.experimental.pallas import tpu_sc as plsc

D_MODEL = 64
D_PAD = 128
SCALE = math.sqrt(D_MODEL)
TBLK = 128
NBUF = 5
L = 16


def _emb_kernel(idx_hbm, tab_hbm, out_hbm, idx_v, gbufs, gsems, ssems,
                *, n_s, nc, nw):
    wid = lax.axis_index("s") * nc + lax.axis_index("c")
    t0 = wid * TBLK
    pltpu.sync_copy(idx_hbm.at[:, pl.ds(t0, TBLK)], idx_v)

    def gather(s, b):
        return pltpu.make_async_copy(
            tab_hbm.at[idx_v.at[s]], gbufs[b], gsems[b])

    def store(s, b):
        return pltpu.make_async_copy(
            gbufs[b], out_hbm.at[s * nw + wid], ssems[b])

    for p in range(4):
        gather(p, p).start()

    n_groups = n_s // NBUF

    def group(g, carry):
        for b in range(NBUF):
            s = g * NBUF + b
            gather(s, b).wait()

            store(s, b).start()

            @pl.when(s >= 1)
            def _wait_prev_store():
                store(s - 1, (b - 1) % NBUF).wait()

            @pl.when(s + 4 < n_s)
            def _fire_next_gather():
                gather(s + 4, (b + 4) % NBUF).start()

        return carry

    lax.fori_loop(0, n_groups, group, 0)
    store(n_s - 1, (n_s - 1) % NBUF).wait()


_VBLK = 8192


def _tpose_scale_body(tabt_ref, out_ref):
    out_ref[:, :D_MODEL] = tabt_ref[...].T * SCALE
    out_ref[:, D_MODEL:] = jnp.zeros((_VBLK, D_PAD - D_MODEL), jnp.float32)


def kernel(inputs, table):
    n_tok, seq = inputs.shape
    info = plsc.get_sparse_core_info()
    nc, ns = info.num_cores, info.num_subcores
    nw = nc * ns
    assert n_tok == nw * TBLK and seq % NBUF == 0

    idx_t = inputs.astype(jnp.int32).T

    vocab = table.shape[0]
    tab_t = table.T
    grid = (vocab + _VBLK - 1) // _VBLK
    tab128 = pl.pallas_call(
        _tpose_scale_body,
        grid=(grid,),
        in_specs=[pl.BlockSpec((D_MODEL, _VBLK), lambda i: (0, i))],
        out_specs=pl.BlockSpec((_VBLK, D_PAD), lambda i: (i, 0)),
        out_shape=jax.ShapeDtypeStruct((vocab, D_PAD), jnp.float32),
    )(tab_t)

    mesh = plsc.VectorSubcoreMesh(core_axis_name="c", subcore_axis_name="s")
    k = functools.partial(
        pl.kernel,
        out_type=jax.ShapeDtypeStruct((seq * nw, TBLK, D_PAD), jnp.float32),
        mesh=mesh,
        scratch_types=[
            pltpu.VMEM((seq, TBLK), jnp.int32),
            [pltpu.VMEM((TBLK, D_PAD), jnp.float32) for _ in range(NBUF)],
            [pltpu.SemaphoreType.DMA for _ in range(NBUF)],
            [pltpu.SemaphoreType.DMA for _ in range(NBUF)],
        ],
    )(functools.partial(_emb_kernel, n_s=seq, nc=nc, nw=nw))

    out = k(idx_t, tab128)
    out = out[:, :, :D_MODEL].reshape(seq, nw * TBLK, D_MODEL)
    return jnp.transpose(out, (1, 0, 2))


# speedup vs baseline: 1.0286x; 1.0286x over previous
import functools
import math

import jax
import jax.numpy as jnp
from jax import lax
from jax.experimental import pallas as pl
from jax.experimental.pallas import tpu as pltpu
from jax.experimental.pallas import tpu_sc as plsc

D_MODEL = 64
D_PAD = 128
SCALE = math.sqrt(D_MODEL)
TBLK = 128
NBUF = 5
L = 16


def _emb_kernel(idx_hbm, tab_hbm, out_hbm, idx_v, gbufs, gsems, ssems,
                *, n_s, nc, nw):
    wid = lax.axis_index("s") * nc + lax.axis_index("c")
    t0 = wid * TBLK
    pltpu.sync_copy(idx_hbm.at[:, pl.ds(t0, TBLK)], idx_v)

    def gather(s, b):
        return pltpu.make_async_copy(
            tab_hbm.at[idx_v.at[s]], gbufs[b], gsems[b])

    def store(s, b):
        return pltpu.make_async_copy(
            gbufs[b], out_hbm.at[s * nw + wid], ssems[b])

    for p in range(4):
        gather(p, p).start()

    n_groups = n_s // NBUF

    def group(g, carry):
        for b in range(NBUF):
            s = g * NBUF + b
            gather(s, b).wait()

            store(s, b).start()

            @pl.when(s >= 1)
            def _wait_prev_store():
                store(s - 1, (b - 1) % NBUF).wait()

            @pl.when(s + 4 < n_s)
            def _fire_next_gather():
                gather(s + 4, (b + 4) % NBUF).start()

        return carry

    lax.fori_loop(0, n_groups, group, 0)
    store(n_s - 1, (n_s - 1) % NBUF).wait()


_VBLK = 16384


def _tpose_scale_body(tabt_ref, out_ref):
    out_ref[:, :D_MODEL] = tabt_ref[...].T * SCALE
    out_ref[:, D_MODEL:] = jnp.zeros((_VBLK, D_PAD - D_MODEL), jnp.float32)


def kernel(inputs, table):
    n_tok, seq = inputs.shape
    info = plsc.get_sparse_core_info()
    nc, ns = info.num_cores, info.num_subcores
    nw = nc * ns
    assert n_tok == nw * TBLK and seq % NBUF == 0

    idx_t = inputs.astype(jnp.int32).T

    vocab = table.shape[0]
    tab_t = table.T
    grid = (vocab + _VBLK - 1) // _VBLK
    tab128 = pl.pallas_call(
        _tpose_scale_body,
        grid=(grid,),
        in_specs=[pl.BlockSpec((D_MODEL, _VBLK), lambda i: (0, i))],
        out_specs=pl.BlockSpec((_VBLK, D_PAD), lambda i: (i, 0)),
        out_shape=jax.ShapeDtypeStruct((vocab, D_PAD), jnp.float32),
    )(tab_t)

    mesh = plsc.VectorSubcoreMesh(core_axis_name="c", subcore_axis_name="s")
    k = functools.partial(
        pl.kernel,
        out_type=jax.ShapeDtypeStruct((seq * nw, TBLK, D_PAD), jnp.float32),
        mesh=mesh,
        scratch_types=[
            pltpu.VMEM((seq, TBLK), jnp.int32),
            [pltpu.VMEM((TBLK, D_PAD), jnp.float32) for _ in range(NBUF)],
            [pltpu.SemaphoreType.DMA for _ in range(NBUF)],
            [pltpu.SemaphoreType.DMA for _ in range(NBUF)],
        ],
    )(functools.partial(_emb_kernel, n_s=seq, nc=nc, nw=nw))

    out = k(idx_t, tab128)
    out = out[:, :, :D_MODEL].reshape(seq, nw * TBLK, D_MODEL)
    return jnp.transpose(out, (1, 0, 2))


# speedup vs baseline: 1.0376x; 1.0088x over previous
import functools
import math

import jax
import jax.numpy as jnp
from jax import lax
from jax.experimental import pallas as pl
from jax.experimental.pallas import tpu as pltpu
from jax.experimental.pallas import tpu_sc as plsc

D_MODEL = 64
D_PAD = 128
SCALE = math.sqrt(D_MODEL)
TBLK = 128
NBUF = 5
L = 16


def _emb_kernel(idx_hbm, tab_hbm, out_hbm, idx_v, gbufs, gsems, ssems,
                *, n_s, nc, nw):
    wid = lax.axis_index("s") * nc + lax.axis_index("c")
    t0 = wid * TBLK
    pltpu.sync_copy(idx_hbm.at[:, pl.ds(t0, TBLK)], idx_v)

    def gather(s, b):
        return pltpu.make_async_copy(
            tab_hbm.at[idx_v.at[s]], gbufs[b], gsems[b])

    def store(s, b):
        return pltpu.make_async_copy(
            gbufs[b], out_hbm.at[s * nw + wid], ssems[b])

    for p in range(4):
        gather(p, p).start()

    n_groups = n_s // NBUF

    def group(g, carry):
        for b in range(NBUF):
            s = g * NBUF + b
            gather(s, b).wait()

            store(s, b).start()

            @pl.when(s >= 1)
            def _wait_prev_store():
                store(s - 1, (b - 1) % NBUF).wait()

            @pl.when(s + 4 < n_s)
            def _fire_next_gather():
                gather(s + 4, (b + 4) % NBUF).start()

        return carry

    lax.fori_loop(0, n_groups, group, 0)
    store(n_s - 1, (n_s - 1) % NBUF).wait()


_VBLK = 32768


def _tpose_scale_body(tabt_ref, out_ref):
    out_ref[:, :D_MODEL] = tabt_ref[...].T * SCALE
    out_ref[:, D_MODEL:] = jnp.zeros((_VBLK, D_PAD - D_MODEL), jnp.float32)


def kernel(inputs, table):
    n_tok, seq = inputs.shape
    info = plsc.get_sparse_core_info()
    nc, ns = info.num_cores, info.num_subcores
    nw = nc * ns
    assert n_tok == nw * TBLK and seq % NBUF == 0

    idx_t = inputs.astype(jnp.int32).T

    vocab = table.shape[0]
    tab_t = table.T
    grid = (vocab + _VBLK - 1) // _VBLK
    tab128 = pl.pallas_call(
        _tpose_scale_body,
        grid=(grid,),
        in_specs=[pl.BlockSpec((D_MODEL, _VBLK), lambda i: (0, i))],
        out_specs=pl.BlockSpec((_VBLK, D_PAD), lambda i: (i, 0)),
        out_shape=jax.ShapeDtypeStruct((vocab, D_PAD), jnp.float32),
    )(tab_t)

    mesh = plsc.VectorSubcoreMesh(core_axis_name="c", subcore_axis_name="s")
    k = functools.partial(
        pl.kernel,
        out_type=jax.ShapeDtypeStruct((seq * nw, TBLK, D_PAD), jnp.float32),
        mesh=mesh,
        scratch_types=[
            pltpu.VMEM((seq, TBLK), jnp.int32),
            [pltpu.VMEM((TBLK, D_PAD), jnp.float32) for _ in range(NBUF)],
            [pltpu.SemaphoreType.DMA for _ in range(NBUF)],
            [pltpu.SemaphoreType.DMA for _ in range(NBUF)],
        ],
    )(functools.partial(_emb_kernel, n_s=seq, nc=nc, nw=nw))

    out = k(idx_t, tab128)
    out = out[:, :, :D_MODEL].reshape(seq, nw * TBLK, D_MODEL)
    return jnp.transpose(out, (1, 0, 2))
